# baseline (speedup 1.0000x reference)
import numpy as np
import jax
import jax.numpy as jnp
from jax.experimental import pallas as pl
from jax.experimental.pallas import tpu as pltpu

B = 1024
T = 60
F = 5
H = 128
N = 2500
CONN = 50
NPAD = 2560
CPAD = 64
BBLK = 256

_PREC = jax.lax.Precision.DEFAULT


def _dot3(a, b):
    ah = a.astype(jnp.bfloat16)
    al = (a - ah.astype(jnp.float32)).astype(jnp.bfloat16)
    bh = b.astype(jnp.bfloat16)
    bl = (b - bh.astype(jnp.float32)).astype(jnp.bfloat16)
    acc = jnp.dot(ah, bh, preferred_element_type=jnp.float32)
    acc += jnp.dot(ah, bl, preferred_element_type=jnp.float32)
    acc += jnp.dot(al, bh, preferred_element_type=jnp.float32)
    return acc


def _sig(x):
    return jax.nn.sigmoid(x)

_GROUPS = [(0, 800, 0), (800, 1500, 1), (1500, 2100, 2), (2100, 2500, 3)]

_msig = np.zeros((NPAD,), np.float32)
_mtanh = np.zeros((NPAD,), np.float32)
_mrelu = np.zeros((NPAD,), np.float32)
_mthr = np.zeros((NPAD,), np.float32)
for _s, _e, _k in _GROUPS:
    if _k in (0, 3):
        _msig[_s:_e] = 1.0
    elif _k == 1:
        _mtanh[_s:_e] = 1.0
    else:
        _mrelu[_s:_e] = 1.0
    if _k in (0, 2):
        _mthr[_s:_e] = 1.0
_MASKS = np.stack([_msig, _mtanh, _mrelu, _mthr], 0)

_GM = np.zeros((NPAD, 128), np.float32)
for _g, (_s, _e, _k) in enumerate(_GROUPS):
    _GM[_s:_e, _g] = 1.0 / (_e - _s)


def _lstm_body(x_ref, wih0_ref, whh0_ref, bih0_ref, bhh0_ref,
               wih1_ref, whh1_ref, bih1_ref, bhh1_ref,
               wp1_ref, bp1_ref, wp2_ref, bp2_ref,
               base_ref, h0, c0, h1, c1):
    t = pl.program_id(0)

    @pl.when(t == 0)
    def _init():
        h0[...] = jnp.zeros_like(h0)
        c0[...] = jnp.zeros_like(c0)
        h1[...] = jnp.zeros_like(h1)
        c1[...] = jnp.zeros_like(c1)

    xt = x_ref[0]
    g0 = (_dot3(xt, wih0_ref[...]) + _dot3(h0[...], whh0_ref[...])
          + bih0_ref[...] + bhh0_ref[...])
    i0 = _sig(g0[:, 0:H])
    f0 = _sig(g0[:, H:2 * H])
    gg0 = jnp.tanh(g0[:, 2 * H:3 * H])
    o0 = _sig(g0[:, 3 * H:4 * H])
    c0n = f0 * c0[...] + i0 * gg0
    h0n = o0 * jnp.tanh(c0n)
    c0[...] = c0n
    h0[...] = h0n

    g1 = (_dot3(h0n, wih1_ref[...]) + _dot3(h1[...], whh1_ref[...])
          + bih1_ref[...] + bhh1_ref[...])
    i1 = _sig(g1[:, 0:H])
    f1g = _sig(g1[:, H:2 * H])
    gg1 = jnp.tanh(g1[:, 2 * H:3 * H])
    o1 = _sig(g1[:, 3 * H:4 * H])
    c1n = f1g * c1[...] + i1 * gg1
    h1n = o1 * jnp.tanh(c1n)
    c1[...] = c1n
    h1[...] = h1n

    @pl.when(t == T - 1)
    def _finish():
        f1 = jax.nn.relu(_dot3(h1n, wp1_ref[...]) + bp1_ref[...])
        base_ref[...] = jnp.tanh(_dot3(f1, wp2_ref[...]) + bp2_ref[...])


def _mbuild_body(idx_ref, w_ref, m_ref):
    m_ref[...] = jnp.zeros_like(m_ref)
    kidx = jax.lax.broadcasted_iota(jnp.int32, (32, NPAD), 0)

    def body(c, _):
        row_i = idx_ref[pl.ds(c, 1), :]
        row_w = w_ref[pl.ds(c, 1), :]
        m_ref[...] += jnp.where(kidx == (row_i & 31), row_w, 0.0)
        return 0

    jax.lax.fori_loop(0, CPAD, body, 0)


def _tail_body(base_ref, m2_ref, sens_ref, thr_ref, masks_ref,
               wa_ref, bwa_ref, wi2_ref, bi2_ref, wi3_ref, bi3_ref,
               wh_ref, bh_ref, out_ref):
    z = _dot3(base_ref[...], m2_ref[...]) * sens_ref[...]
    zz = z - thr_ref[...] * masks_ref[3:4, :]
    all_out = (masks_ref[0:1, :] * _sig(zz)
               + masks_ref[1:2, :] * jnp.tanh(zz)
               + masks_ref[2:3, :] * jax.nn.relu(zz))
    y = _dot3(all_out, wa_ref[...]) + bwa_ref[...]
    g1 = jax.nn.relu(y[:, 0:256])
    acts = y[:, 256:260]
    g2 = jax.nn.relu(_dot3(g1, wi2_ref[...]) + bi2_ref[...])
    integ = jnp.tanh(_dot3(g2, wi3_ref[...]) + bi3_ref[...])
    hl = _dot3(integ, wh_ref[...]) + bh_ref[...]
    overall = _sig(hl[:, 14:15])
    out_ref[...] = jnp.concatenate([hl[:, 0:15], overall, acts], axis=1)


def kernel(x, W_ih0, W_hh0, b_ih0, b_hh0, W_ih1, W_hh1, b_ih1, b_hh1,
           Wp1, bp1, Wp2, bp2, sensitivity, threshold, conn_w,
           Wi1, bi1, Wi2, bi2, Wi3, bi3, Wt, bt, Wpat, bpat, Wk, bk,
           Wv, bv, Wc, bc, conn_idx):
    f32 = jnp.float32
    G4 = 4 * H

    xT = jnp.pad(jnp.transpose(x, (1, 0, 2)), ((0, 0), (0, 0), (0, 8 - F)))
    wih0T = jnp.pad(W_ih0, ((0, 0), (0, 8 - F))).T
    whh0T = W_hh0.T
    wih1T = W_ih1.T
    whh1T = W_hh1.T
    bih0r = b_ih0.reshape(1, G4)
    bhh0r = b_hh0.reshape(1, G4)
    bih1r = b_ih1.reshape(1, G4)
    bhh1r = b_hh1.reshape(1, G4)
    wp1T = Wp1.T
    bp1r = bp1.reshape(1, 64)
    wp2T = Wp2.T
    bp2r = bp2.reshape(1, 32)

    idxT = jnp.pad(conn_idx, ((0, NPAD - N), (0, CPAD - CONN))).T
    wT = jnp.pad(conn_w, ((0, NPAD - N), (0, CPAD - CONN))).T
    sens_p = jnp.pad(sensitivity, (0, NPAD - N)).reshape(1, NPAD)
    thr_p = jnp.pad(threshold, (0, NPAD - N)).reshape(1, NPAD)
    masks = jnp.asarray(_MASKS)

    wi1T = jnp.pad(Wi1, ((0, 0), (0, NPAD - N))).T
    wa = jnp.concatenate([wi1T, jnp.asarray(_GM)], axis=1)
    bwa = jnp.concatenate([bi1, jnp.zeros((128,), f32)]).reshape(1, 384)
    wi2T = Wi2.T
    bi2r = bi2.reshape(1, 64)
    wi3T = Wi3.T
    bi3r = bi3.reshape(1, 32)
    wh = jnp.concatenate([Wt, Wpat, Wk, Wv, Wc], axis=0)
    whT = jnp.pad(wh, ((0, 17), (0, 0))).T
    bh = jnp.pad(jnp.concatenate([bt, bpat, bk, bv, bc]), (0, 17))
    bh = bh.reshape(1, 32)

    cst = lambda shape: pl.BlockSpec(shape, lambda t: tuple(0 for _ in shape))
    base = pl.pallas_call(
        _lstm_body,
        grid=(T,),
        in_specs=[
            pl.BlockSpec((1, B, 8), lambda t: (t, 0, 0)),
            cst((8, G4)), cst((H, G4)), cst((1, G4)), cst((1, G4)),
            cst((H, G4)), cst((H, G4)), cst((1, G4)), cst((1, G4)),
            cst((H, 64)), cst((1, 64)), cst((64, 32)), cst((1, 32)),
        ],
        out_specs=pl.BlockSpec((B, 32), lambda t: (0, 0)),
        out_shape=jax.ShapeDtypeStruct((B, 32), f32),
        scratch_shapes=[pltpu.VMEM((B, H), f32) for _ in range(4)],
    )(xT, wih0T, whh0T, bih0r, bhh0r, wih1T, whh1T, bih1r, bhh1r,
      wp1T, bp1r, wp2T, bp2r)

    m2 = pl.pallas_call(
        _mbuild_body,
        out_shape=jax.ShapeDtypeStruct((32, NPAD), f32),
    )(idxT, wT)

    nb = B // BBLK
    cst2 = cst
    out2 = pl.pallas_call(
        _tail_body,
        grid=(nb,),
        in_specs=[
            pl.BlockSpec((BBLK, 32), lambda b: (b, 0)),
            cst2((32, NPAD)), cst2((1, NPAD)), cst2((1, NPAD)),
            cst2((4, NPAD)), cst2((NPAD, 384)), cst2((1, 384)),
            cst2((256, 64)), cst2((1, 64)), cst2((64, 32)), cst2((1, 32)),
            cst2((32, 32)), cst2((1, 32)),
        ],
        out_specs=pl.BlockSpec((BBLK, 20), lambda b: (b, 0)),
        out_shape=jax.ShapeDtypeStruct((B, 20), f32),
    )(base, m2, sens_p, thr_p, masks, wa, bwa, wi2T, bi2r, wi3T, bi3r,
      whT, bh)

    trend = out2[:, 0:3]
    patt = out2[:, 3:9]
    keyl = out2[:, 9:13]
    vol = out2[:, 13:14]
    conf = out2[:, 14:15]
    overall = out2[:, 15]
    a0 = out2[:, 16]
    a1 = out2[:, 17]
    a2 = out2[:, 18]
    a3 = out2[:, 19]
    return (trend, patt, keyl, vol, conf, overall, a0, a1, a2, a3)


# speedup vs baseline: 1.1764x; 1.1764x over previous
import numpy as np
import jax
import jax.numpy as jnp
from jax.experimental import pallas as pl
from jax.experimental.pallas import tpu as pltpu

B = 1024
T = 60
F = 5
H = 128
N = 2500
CONN = 50
NPAD = 2560
CPAD = 64
BBLK = 256

_PREC = jax.lax.Precision.DEFAULT


def _dot3(a, b):
    ah = a.astype(jnp.bfloat16)
    al = (a - ah.astype(jnp.float32)).astype(jnp.bfloat16)
    bh = b.astype(jnp.bfloat16)
    bl = (b - bh.astype(jnp.float32)).astype(jnp.bfloat16)
    acc = jnp.dot(ah, bh, preferred_element_type=jnp.float32)
    acc += jnp.dot(ah, bl, preferred_element_type=jnp.float32)
    acc += jnp.dot(al, bh, preferred_element_type=jnp.float32)
    return acc


def _sig(x):
    return jax.nn.sigmoid(x)


def _dot3w(a, bh, bl):
    ah = a.astype(jnp.bfloat16)
    al = (a - ah.astype(jnp.float32)).astype(jnp.bfloat16)
    return (jnp.dot(ah, bh, preferred_element_type=jnp.float32)
            + jnp.dot(ah, bl, preferred_element_type=jnp.float32)
            + jnp.dot(al, bh, preferred_element_type=jnp.float32))


def _split_hl(w):
    hi = w.astype(jnp.bfloat16)
    lo = (w - hi.astype(jnp.float32)).astype(jnp.bfloat16)
    return hi, lo

_GROUPS = [(0, 800, 0), (800, 1500, 1), (1500, 2100, 2), (2100, 2500, 3)]

_msig = np.zeros((NPAD,), np.float32)
_mtanh = np.zeros((NPAD,), np.float32)
_mrelu = np.zeros((NPAD,), np.float32)
_mthr = np.zeros((NPAD,), np.float32)
for _s, _e, _k in _GROUPS:
    if _k in (0, 3):
        _msig[_s:_e] = 1.0
    elif _k == 1:
        _mtanh[_s:_e] = 1.0
    else:
        _mrelu[_s:_e] = 1.0
    if _k in (0, 2):
        _mthr[_s:_e] = 1.0
_MASKS = np.stack([_msig, _mtanh, _mrelu, _mthr], 0)

_GM = np.zeros((NPAD, 128), np.float32)
for _g, (_s, _e, _k) in enumerate(_GROUPS):
    _GM[_s:_e, _g] = 1.0 / (_e - _s)


def _lstm_body(x_ref, wih0h_ref, wih0l_ref, whh0h_ref, whh0l_ref, b0_ref,
               w1h_ref, w1l_ref, b1_ref,
               wp1_ref, bp1_ref, wp2_ref, bp2_ref,
               base_ref, h0, c0, h1, c1):
    t = pl.program_id(0)

    @pl.when(t == 0)
    def _init():
        h0[...] = jnp.zeros_like(h0)
        c0[...] = jnp.zeros_like(c0)
        h1[...] = jnp.zeros_like(h1)
        c1[...] = jnp.zeros_like(c1)

    xt = x_ref[0]
    g0 = (_dot3w(xt, wih0h_ref[...], wih0l_ref[...])
          + _dot3w(h0[...], whh0h_ref[...], whh0l_ref[...])
          + b0_ref[...])
    i0 = _sig(g0[:, 0:H])
    f0 = _sig(g0[:, H:2 * H])
    gg0 = jnp.tanh(g0[:, 2 * H:3 * H])
    o0 = _sig(g0[:, 3 * H:4 * H])
    c0n = f0 * c0[...] + i0 * gg0
    h0n = o0 * jnp.tanh(c0n)
    c0[...] = c0n
    h0[...] = h0n

    a1 = jnp.concatenate([h0n, h1[...]], axis=1)
    g1 = _dot3w(a1, w1h_ref[...], w1l_ref[...]) + b1_ref[...]
    i1 = _sig(g1[:, 0:H])
    f1g = _sig(g1[:, H:2 * H])
    gg1 = jnp.tanh(g1[:, 2 * H:3 * H])
    o1 = _sig(g1[:, 3 * H:4 * H])
    c1n = f1g * c1[...] + i1 * gg1
    h1n = o1 * jnp.tanh(c1n)
    c1[...] = c1n
    h1[...] = h1n

    @pl.when(t == T - 1)
    def _finish():
        f1 = jax.nn.relu(_dot3(h1n, wp1_ref[...]) + bp1_ref[...])
        base_ref[...] = jnp.tanh(_dot3(f1, wp2_ref[...]) + bp2_ref[...])


def _mbuild_body(idx_ref, w_ref, m_ref):
    m_ref[...] = jnp.zeros_like(m_ref)
    kidx = jax.lax.broadcasted_iota(jnp.int32, (32, NPAD), 0)

    def body(c, _):
        row_i = idx_ref[pl.ds(c, 1), :]
        row_w = w_ref[pl.ds(c, 1), :]
        m_ref[...] += jnp.where(kidx == (row_i & 31), row_w, 0.0)
        return 0

    jax.lax.fori_loop(0, CPAD, body, 0)


def _tail_body(base_ref, m2_ref, sens_ref, thr_ref, masks_ref,
               wa_ref, bwa_ref, wi2_ref, bi2_ref, wi3_ref, bi3_ref,
               wh_ref, bh_ref, out_ref):
    z = _dot3(base_ref[...], m2_ref[...]) * sens_ref[...]
    zz = z - thr_ref[...] * masks_ref[3:4, :]
    all_out = (masks_ref[0:1, :] * _sig(zz)
               + masks_ref[1:2, :] * jnp.tanh(zz)
               + masks_ref[2:3, :] * jax.nn.relu(zz))
    y = _dot3(all_out, wa_ref[...]) + bwa_ref[...]
    g1 = jax.nn.relu(y[:, 0:256])
    acts = y[:, 256:260]
    g2 = jax.nn.relu(_dot3(g1, wi2_ref[...]) + bi2_ref[...])
    integ = jnp.tanh(_dot3(g2, wi3_ref[...]) + bi3_ref[...])
    hl = _dot3(integ, wh_ref[...]) + bh_ref[...]
    overall = _sig(hl[:, 14:15])
    out_ref[...] = jnp.concatenate([hl[:, 0:15], overall, acts], axis=1)


def kernel(x, W_ih0, W_hh0, b_ih0, b_hh0, W_ih1, W_hh1, b_ih1, b_hh1,
           Wp1, bp1, Wp2, bp2, sensitivity, threshold, conn_w,
           Wi1, bi1, Wi2, bi2, Wi3, bi3, Wt, bt, Wpat, bpat, Wk, bk,
           Wv, bv, Wc, bc, conn_idx):
    f32 = jnp.float32
    G4 = 4 * H

    xT = jnp.pad(jnp.transpose(x, (1, 0, 2)), ((0, 0), (0, 0), (0, 8 - F)))
    wih0h, wih0l = _split_hl(jnp.pad(W_ih0, ((0, 0), (0, 8 - F))).T)
    whh0h, whh0l = _split_hl(W_hh0.T)
    w1h, w1l = _split_hl(jnp.concatenate([W_ih1.T, W_hh1.T], axis=0))
    b0r = (b_ih0 + b_hh0).reshape(1, G4)
    b1r = (b_ih1 + b_hh1).reshape(1, G4)
    wp1T = Wp1.T
    bp1r = bp1.reshape(1, 64)
    wp2T = Wp2.T
    bp2r = bp2.reshape(1, 32)

    idxT = jnp.pad(conn_idx, ((0, NPAD - N), (0, CPAD - CONN))).T
    wT = jnp.pad(conn_w, ((0, NPAD - N), (0, CPAD - CONN))).T
    sens_p = jnp.pad(sensitivity, (0, NPAD - N)).reshape(1, NPAD)
    thr_p = jnp.pad(threshold, (0, NPAD - N)).reshape(1, NPAD)
    masks = jnp.asarray(_MASKS)

    wi1T = jnp.pad(Wi1, ((0, 0), (0, NPAD - N))).T
    wa = jnp.concatenate([wi1T, jnp.asarray(_GM)], axis=1)
    bwa = jnp.concatenate([bi1, jnp.zeros((128,), f32)]).reshape(1, 384)
    wi2T = Wi2.T
    bi2r = bi2.reshape(1, 64)
    wi3T = Wi3.T
    bi3r = bi3.reshape(1, 32)
    wh = jnp.concatenate([Wt, Wpat, Wk, Wv, Wc], axis=0)
    whT = jnp.pad(wh, ((0, 17), (0, 0))).T
    bh = jnp.pad(jnp.concatenate([bt, bpat, bk, bv, bc]), (0, 17))
    bh = bh.reshape(1, 32)

    cst = lambda shape: pl.BlockSpec(shape, lambda t: tuple(0 for _ in shape))
    base = pl.pallas_call(
        _lstm_body,
        grid=(T,),
        in_specs=[
            pl.BlockSpec((1, B, 8), lambda t: (t, 0, 0)),
            cst((8, G4)), cst((8, G4)), cst((H, G4)), cst((H, G4)),
            cst((1, G4)),
            cst((2 * H, G4)), cst((2 * H, G4)), cst((1, G4)),
            cst((H, 64)), cst((1, 64)), cst((64, 32)), cst((1, 32)),
        ],
        out_specs=pl.BlockSpec((B, 32), lambda t: (0, 0)),
        out_shape=jax.ShapeDtypeStruct((B, 32), f32),
        scratch_shapes=[pltpu.VMEM((B, H), f32) for _ in range(4)],
    )(xT, wih0h, wih0l, whh0h, whh0l, b0r,
      w1h, w1l, b1r,
      wp1T, bp1r, wp2T, bp2r)

    m2 = pl.pallas_call(
        _mbuild_body,
        out_shape=jax.ShapeDtypeStruct((32, NPAD), f32),
    )(idxT, wT)

    nb = B // BBLK
    cst2 = cst
    out2 = pl.pallas_call(
        _tail_body,
        grid=(nb,),
        in_specs=[
            pl.BlockSpec((BBLK, 32), lambda b: (b, 0)),
            cst2((32, NPAD)), cst2((1, NPAD)), cst2((1, NPAD)),
            cst2((4, NPAD)), cst2((NPAD, 384)), cst2((1, 384)),
            cst2((256, 64)), cst2((1, 64)), cst2((64, 32)), cst2((1, 32)),
            cst2((32, 32)), cst2((1, 32)),
        ],
        out_specs=pl.BlockSpec((BBLK, 20), lambda b: (b, 0)),
        out_shape=jax.ShapeDtypeStruct((B, 20), f32),
    )(base, m2, sens_p, thr_p, masks, wa, bwa, wi2T, bi2r, wi3T, bi3r,
      whT, bh)

    trend = out2[:, 0:3]
    patt = out2[:, 3:9]
    keyl = out2[:, 9:13]
    vol = out2[:, 13:14]
    conf = out2[:, 14:15]
    overall = out2[:, 15]
    a0 = out2[:, 16]
    a1 = out2[:, 17]
    a2 = out2[:, 18]
    a3 = out2[:, 19]
    return (trend, patt, keyl, vol, conf, overall, a0, a1, a2, a3)


# speedup vs baseline: 1.7674x; 1.5024x over previous
import functools
import numpy as np
import jax
import jax.numpy as jnp
from jax import lax
from jax.experimental import pallas as pl
from jax.experimental.pallas import tpu as pltpu
from jax.experimental.pallas import tpu_sc as plsc

B = 1024
T = 60
F = 5
H = 128
N = 2500
CONN = 50
NPAD = 2560
CPAD = 64
BBLK = 256

_PREC = jax.lax.Precision.DEFAULT


def _dot3(a, b):
    return jnp.dot(a, b, preferred_element_type=jnp.float32)


def _sig(x):
    return jax.nn.sigmoid(x)


def _tanh(x):
    return jnp.tanh(x)


def _dot3w(a, bh, bl):
    ah = a.astype(jnp.bfloat16)
    al = (a - ah.astype(jnp.float32)).astype(jnp.bfloat16)
    return (jnp.dot(ah, bh, preferred_element_type=jnp.float32)
            + jnp.dot(ah, bl, preferred_element_type=jnp.float32)
            + jnp.dot(al, bh, preferred_element_type=jnp.float32))


def _split_hl(w):
    hi = w.astype(jnp.bfloat16)
    lo = (w - hi.astype(jnp.float32)).astype(jnp.bfloat16)
    return hi, lo

_GROUPS = [(0, 800, 0), (800, 1500, 1), (1500, 2100, 2), (2100, 2500, 3)]

_msig = np.zeros((NPAD,), np.float32)
_mtanh = np.zeros((NPAD,), np.float32)
_mrelu = np.zeros((NPAD,), np.float32)
_mthr = np.zeros((NPAD,), np.float32)
for _s, _e, _k in _GROUPS:
    if _k in (0, 3):
        _msig[_s:_e] = 1.0
    elif _k == 1:
        _mtanh[_s:_e] = 1.0
    else:
        _mrelu[_s:_e] = 1.0
    if _k in (0, 2):
        _mthr[_s:_e] = 1.0
_MASKS = np.stack([_msig, _mtanh, _mrelu, _mthr], 0)

_GM = np.zeros((NPAD, 128), np.float32)
for _g, (_s, _e, _k) in enumerate(_GROUPS):
    _GM[_s:_e, _g] = 1.0 / (_e - _s)


def _lstm_body(x_ref, wih0_ref, whh0_ref, b0_ref,
               wih1_ref, whh1_ref, b1_ref,
               wp1_ref, bp1_ref, wp2_ref, bp2_ref,
               base_ref, h0, c0, h1, c1):
    t = pl.program_id(0)

    @pl.when(t == 0)
    def _init():
        h0[...] = jnp.zeros_like(h0)
        c0[...] = jnp.zeros_like(c0)
        h1[...] = jnp.zeros_like(h1)
        c1[...] = jnp.zeros_like(c1)

    xt = x_ref[0]
    g0 = (_dot3(xt, wih0_ref[...]) + _dot3(h0[...], whh0_ref[...])
          + b0_ref[...])
    i0 = _sig(g0[:, 0:H])
    f0 = _sig(g0[:, H:2 * H])
    gg0 = _tanh(g0[:, 2 * H:3 * H])
    o0 = _sig(g0[:, 3 * H:4 * H])
    c0n = f0 * c0[...] + i0 * gg0
    h0n = o0 * _tanh(c0n)
    c0[...] = c0n
    h0[...] = h0n

    g1 = (_dot3(h0n, wih1_ref[...]) + _dot3(h1[...], whh1_ref[...])
          + b1_ref[...])
    i1 = _sig(g1[:, 0:H])
    f1g = _sig(g1[:, H:2 * H])
    gg1 = _tanh(g1[:, 2 * H:3 * H])
    o1 = _sig(g1[:, 3 * H:4 * H])
    c1n = f1g * c1[...] + i1 * gg1
    h1n = o1 * _tanh(c1n)
    c1[...] = c1n
    h1[...] = h1n

    @pl.when(t == T - 1)
    def _finish():
        f1 = jax.nn.relu(_dot3(h1n, wp1_ref[...]) + bp1_ref[...])
        base_ref[...] = _tanh(_dot3(f1, wp2_ref[...]) + bp2_ref[...])


_NW = 32
_ROWS_W = NPAD // _NW
_ELEMS_W = _ROWS_W * CPAD
_NCHUNK = _ELEMS_W // 16


def _msc_body(idx_hbm, w_hbm, m_hbm, idxv, wv, mloc):
    cid = lax.axis_index("c")
    sid = lax.axis_index("s")
    wid = sid * 2 + cid
    base = wid * _ELEMS_W
    pltpu.sync_copy(idx_hbm.at[pl.ds(base, _ELEMS_W)], idxv)
    pltpu.sync_copy(w_hbm.at[pl.ds(base, _ELEMS_W)], wv)

    def zero_body(j, _):
        mloc[pl.ds(j * 16, 16)] = jnp.zeros((16,), jnp.float32)
        return 0

    lax.fori_loop(0, (_ROWS_W * 32) // 16, zero_body, 0)

    def scat_body(k, _):
        lanes = lax.iota(jnp.int32, 16)
        e = k * 16 + lanes
        nloc = e >> 6
        iv = idxv[pl.ds(k * 16, 16)]
        vv = wv[pl.ds(k * 16, 16)]
        dv = (iv & 31) * _ROWS_W + nloc
        plsc.addupdate_scatter(mloc, [dv], vv)
        return 0

    lax.fori_loop(0, _NCHUNK, scat_body, 0)

    def out_body(k, _):
        pltpu.sync_copy(mloc.at[pl.ds(k * _ROWS_W, _ROWS_W)],
                        m_hbm.at[pl.ds(k * NPAD + wid * _ROWS_W, _ROWS_W)])
        return 0

    lax.fori_loop(0, 32, out_body, 0)


def _msc_call(idx_flat, w_flat):
    call = functools.partial(
        pl.kernel,
        mesh=plsc.VectorSubcoreMesh(core_axis_name="c", subcore_axis_name="s"),
        out_type=jax.ShapeDtypeStruct((32 * NPAD,), jnp.float32),
        scratch_types=[
            pltpu.VMEM((_ELEMS_W,), jnp.int32),
            pltpu.VMEM((_ELEMS_W,), jnp.float32),
            pltpu.VMEM((_ROWS_W * 32,), jnp.float32),
        ],
    )(_msc_body)
    return call(idx_flat, w_flat)


def _mbuild_body(idx_ref, w_ref, m_ref):
    m_ref[...] = jnp.zeros_like(m_ref)
    kidx = jax.lax.broadcasted_iota(jnp.int32, (32, NPAD), 0)

    def body(c, _):
        row_i = idx_ref[pl.ds(c, 1), :]
        row_w = w_ref[pl.ds(c, 1), :]
        m_ref[...] += jnp.where(kidx == (row_i & 31), row_w, 0.0)
        return 0

    jax.lax.fori_loop(0, CPAD, body, 0)


def _tail_body(base_ref, m2_ref, sens_ref, thr_ref, masks_ref,
               wa_ref, bwa_ref, wi2_ref, bi2_ref, wi3_ref, bi3_ref,
               wh_ref, bh_ref, out_ref):
    z = _dot3(base_ref[...], m2_ref[...]) * sens_ref[...]
    zz = z - thr_ref[...] * masks_ref[3:4, :]
    all_out = (masks_ref[0:1, :] * _sig(zz)
               + masks_ref[1:2, :] * _tanh(zz)
               + masks_ref[2:3, :] * jax.nn.relu(zz))
    y = _dot3(all_out, wa_ref[...]) + bwa_ref[...]
    g1 = jax.nn.relu(y[:, 0:256])
    acts = y[:, 256:260]
    g2 = jax.nn.relu(_dot3(g1, wi2_ref[...]) + bi2_ref[...])
    integ = _tanh(_dot3(g2, wi3_ref[...]) + bi3_ref[...])
    hl = _dot3(integ, wh_ref[...]) + bh_ref[...]
    overall = _sig(hl[:, 14:15])
    out_ref[...] = jnp.concatenate([hl[:, 0:15], overall, acts], axis=1)


def kernel(x, W_ih0, W_hh0, b_ih0, b_hh0, W_ih1, W_hh1, b_ih1, b_hh1,
           Wp1, bp1, Wp2, bp2, sensitivity, threshold, conn_w,
           Wi1, bi1, Wi2, bi2, Wi3, bi3, Wt, bt, Wpat, bpat, Wk, bk,
           Wv, bv, Wc, bc, conn_idx):
    f32 = jnp.float32
    G4 = 4 * H

    xT = jnp.pad(jnp.transpose(x, (1, 0, 2)), ((0, 0), (0, 0), (0, 8 - F)))
    wih0T = jnp.pad(W_ih0, ((0, 0), (0, 8 - F))).T
    whh0T = W_hh0.T
    wih1T = W_ih1.T
    whh1T = W_hh1.T
    b0r = (b_ih0 + b_hh0).reshape(1, G4)
    b1r = (b_ih1 + b_hh1).reshape(1, G4)
    wp1T = Wp1.T
    bp1r = bp1.reshape(1, 64)
    wp2T = Wp2.T
    bp2r = bp2.reshape(1, 32)

    idxT = jnp.pad(conn_idx, ((0, NPAD - N), (0, CPAD - CONN))).T
    wT = jnp.pad(conn_w, ((0, NPAD - N), (0, CPAD - CONN))).T
    sens_p = jnp.pad(sensitivity, (0, NPAD - N)).reshape(1, NPAD)
    thr_p = jnp.pad(threshold, (0, NPAD - N)).reshape(1, NPAD)
    masks = jnp.asarray(_MASKS)

    wi1T = jnp.pad(Wi1, ((0, 0), (0, NPAD - N))).T
    wa = jnp.concatenate([wi1T, jnp.asarray(_GM)], axis=1)
    bwa = jnp.concatenate([bi1, jnp.zeros((128,), f32)]).reshape(1, 384)
    wi2T = Wi2.T
    bi2r = bi2.reshape(1, 64)
    wi3T = Wi3.T
    bi3r = bi3.reshape(1, 32)
    wh = jnp.concatenate([Wt, Wpat, Wk, Wv, Wc], axis=0)
    whT = jnp.pad(wh, ((0, 17), (0, 0))).T
    bh = jnp.pad(jnp.concatenate([bt, bpat, bk, bv, bc]), (0, 17))
    bh = bh.reshape(1, 32)

    cst = lambda shape: pl.BlockSpec(shape, lambda t: tuple(0 for _ in shape))
    base = pl.pallas_call(
        _lstm_body,
        grid=(T,),
        in_specs=[
            pl.BlockSpec((1, B, 8), lambda t: (t, 0, 0)),
            cst((8, G4)), cst((H, G4)), cst((1, G4)),
            cst((H, G4)), cst((H, G4)), cst((1, G4)),
            cst((H, 64)), cst((1, 64)), cst((64, 32)), cst((1, 32)),
        ],
        out_specs=pl.BlockSpec((B, 32), lambda t: (0, 0)),
        out_shape=jax.ShapeDtypeStruct((B, 32), f32),
        scratch_shapes=[pltpu.VMEM((B, H), f32) for _ in range(4)],
    )(xT, wih0T, whh0T, b0r,
      wih1T, whh1T, b1r,
      wp1T, bp1r, wp2T, bp2r)

    m2 = pl.pallas_call(
        _mbuild_body,
        out_shape=jax.ShapeDtypeStruct((32, NPAD), f32),
    )(idxT, wT)

    nb = B // BBLK
    cst2 = cst
    out2 = pl.pallas_call(
        _tail_body,
        grid=(nb,),
        in_specs=[
            pl.BlockSpec((BBLK, 32), lambda b: (b, 0)),
            cst2((32, NPAD)), cst2((1, NPAD)), cst2((1, NPAD)),
            cst2((4, NPAD)), cst2((NPAD, 384)), cst2((1, 384)),
            cst2((256, 64)), cst2((1, 64)), cst2((64, 32)), cst2((1, 32)),
            cst2((32, 32)), cst2((1, 32)),
        ],
        out_specs=pl.BlockSpec((BBLK, 20), lambda b: (b, 0)),
        out_shape=jax.ShapeDtypeStruct((B, 20), f32),
    )(base, m2, sens_p, thr_p, masks, wa, bwa, wi2T, bi2r, wi3T, bi3r,
      whT, bh)

    trend = out2[:, 0:3]
    patt = out2[:, 3:9]
    keyl = out2[:, 9:13]
    vol = out2[:, 13:14]
    conf = out2[:, 14:15]
    overall = out2[:, 15]
    a0 = out2[:, 16]
    a1 = out2[:, 17]
    a2 = out2[:, 18]
    a3 = out2[:, 19]
    return (trend, patt, keyl, vol, conf, overall, a0, a1, a2, a3)
